# baseline (speedup 1.0000x reference)
import jax
import jax.numpy as jnp
from jax.experimental import pallas as pl
from jax.experimental.pallas import tpu as pltpu

_BN = 1024


def _prep_body(std_col_ref, mu_ref, wt_ref, msqh_ref):
    inv_col = 1.0 / std_col_ref[...]
    mu_t = jnp.swapaxes(mu_ref[...], 0, 1)
    muw_t = mu_t * inv_col
    wt_ref[...] = muw_t
    msqh_ref[...] = 0.5 * jnp.sum(mu_t * muw_t, axis=0, keepdims=True)


def _gauss_body(std_row_ref, wt_ref, msqh_ref, x_ref, out_ref):
    inv_row = 1.0 / std_row_ref[...]
    xb = x_ref[...]
    xsq_half = 0.5 * jnp.sum(xb * xb * inv_row, axis=1, keepdims=True)
    cross = jnp.dot(xb, wt_ref[...], preferred_element_type=jnp.float32)
    out_ref[...] = jnp.exp(cross - xsq_half - msqh_ref[...])


def kernel(x, mu, std):
    n, d = x.shape
    k, nc, _ = mu.shape
    mu2d = mu.reshape(k, nc * d)
    std_row = std.reshape(1, d)
    std_col = std.reshape(d, 1)

    muw_t, msq_half = pl.pallas_call(
        _prep_body,
        grid=(1,),
        in_specs=[
            pl.BlockSpec((d, 1), lambda i: (0, 0)),
            pl.BlockSpec((k, d), lambda i: (0, 0)),
        ],
        out_specs=[
            pl.BlockSpec((d, k), lambda i: (0, 0)),
            pl.BlockSpec((1, k), lambda i: (0, 0)),
        ],
        out_shape=[
            jax.ShapeDtypeStruct((d, k), jnp.float32),
            jax.ShapeDtypeStruct((1, k), jnp.float32),
        ],
        compiler_params=pltpu.CompilerParams(
            vmem_limit_bytes=60 * 1024 * 1024,
        ),
    )(std_col, mu2d)

    return pl.pallas_call(
        _gauss_body,
        grid=(n // _BN,),
        in_specs=[
            pl.BlockSpec((1, d), lambda i: (0, 0)),
            pl.BlockSpec((d, k), lambda i: (0, 0)),
            pl.BlockSpec((1, k), lambda i: (0, 0)),
            pl.BlockSpec((_BN, d), lambda i: (i, 0)),
        ],
        out_specs=pl.BlockSpec((_BN, k), lambda i: (i, 0)),
        out_shape=jax.ShapeDtypeStruct((n, k), jnp.float32),
        compiler_params=pltpu.CompilerParams(
            dimension_semantics=("parallel",),
            vmem_limit_bytes=60 * 1024 * 1024,
        ),
    )(std_row, muw_t, msq_half, x)


# speedup vs baseline: 1.1547x; 1.1547x over previous
import jax
import jax.numpy as jnp
from jax.experimental import pallas as pl
from jax.experimental.pallas import tpu as pltpu

_BN = 1024


def _gauss_body(std_row_ref, mu_ref, x_ref, out_ref):
    inv_row = 1.0 / std_row_ref[...]
    mu0 = mu_ref[...]
    muw = mu0 * inv_row
    msq_half = 0.5 * jax.lax.dot_general(
        inv_row, mu0 * mu0,
        dimension_numbers=(((1,), (1,)), ((), ())),
        preferred_element_type=jnp.float32)
    xb = x_ref[...]
    xsq_half = 0.5 * jnp.sum(xb * xb * inv_row, axis=1, keepdims=True)
    cross = jax.lax.dot_general(
        xb, muw,
        dimension_numbers=(((1,), (1,)), ((), ())),
        preferred_element_type=jnp.float32)
    out_ref[...] = jnp.exp(cross - xsq_half - msq_half)


def kernel(x, mu, std):
    n, d = x.shape
    k = mu.shape[0]
    mu0 = mu[:, 0, :]
    std_row = std.reshape(1, d)
    return pl.pallas_call(
        _gauss_body,
        grid=(n // _BN,),
        in_specs=[
            pl.BlockSpec((1, d), lambda i: (0, 0)),
            pl.BlockSpec((k, d), lambda i: (0, 0)),
            pl.BlockSpec((_BN, d), lambda i: (i, 0)),
        ],
        out_specs=pl.BlockSpec((_BN, k), lambda i: (i, 0)),
        out_shape=jax.ShapeDtypeStruct((n, k), jnp.float32),
        compiler_params=pltpu.CompilerParams(
            dimension_semantics=("parallel",),
            vmem_limit_bytes=60 * 1024 * 1024,
        ),
    )(std_row, mu0, x)
